# baseline (speedup 1.0000x reference)
import jax
import jax.numpy as jnp
from jax.experimental import pallas as pl

H = 384
W = 384
HW = H * W
D = 256
NP = 10240
E = 80000
NEG = 0.01

_INTERPRET = False


def _leaky(x):
    return jnp.where(x >= 0, x, NEG * x)


def _pc(body, grid, in_specs, out_specs, out_shape):
    return pl.pallas_call(body, grid=grid, in_specs=in_specs,
                          out_specs=out_specs, out_shape=out_shape,
                          interpret=_INTERPRET)


def _mm_leaky_body(x_ref, w_ref, b_ref, o_ref):
    o_ref[...] = _leaky(jnp.dot(x_ref[...], w_ref[...],
                                preferred_element_type=jnp.float32) + b_ref[...])


def _mm_leaky(x, w, b, bm):
    M, K = x.shape
    N = w.shape[1]
    return _pc(
        _mm_leaky_body, (M // bm,),
        [pl.BlockSpec((bm, K), lambda i: (i, 0)),
         pl.BlockSpec((K, N), lambda i: (0, 0)),
         pl.BlockSpec((1, N), lambda i: (0, 0))],
        pl.BlockSpec((bm, N), lambda i: (i, 0)),
        jax.ShapeDtypeStruct((M, N), jnp.float32),
    )(x, w, b.reshape(1, N))


def _segdiv_body(s_ref, c_ref, o_ref):
    o_ref[...] = s_ref[...] / jnp.maximum(c_ref[...], 1.0)


def _segdiv(s, c, bm):
    M, N = s.shape
    return _pc(
        _segdiv_body, (M // bm,),
        [pl.BlockSpec((bm, N), lambda i: (i, 0)),
         pl.BlockSpec((bm, 1), lambda i: (i, 0))],
        pl.BlockSpec((bm, N), lambda i: (i, 0)),
        jax.ShapeDtypeStruct((M, N), jnp.float32),
    )(s, c.reshape(M, 1))


def _node_body(x_ref, s_ref, sang_ref, deg_ref, ws_ref, bs_ref, wmx_ref,
               wma_ref, bmv_ref, o_ref):
    agg = jnp.dot(s_ref[...], wmx_ref[...], preferred_element_type=jnp.float32)
    agg = agg + sang_ref[...] * wma_ref[...] + deg_ref[...] * bmv_ref[...]
    agg = agg / jnp.maximum(deg_ref[...], 1.0)
    y = jnp.dot(x_ref[...], ws_ref[...], preferred_element_type=jnp.float32)
    o_ref[...] = _leaky(y + bs_ref[...] + agg)


def _node_update(x, s, sang, deg, ws, bs, wm, bmv, bm_blk=512):
    M = x.shape[0]
    return _pc(
        _node_body, (M // bm_blk,),
        [pl.BlockSpec((bm_blk, D), lambda i: (i, 0)),
         pl.BlockSpec((bm_blk, D), lambda i: (i, 0)),
         pl.BlockSpec((bm_blk, 1), lambda i: (i, 0)),
         pl.BlockSpec((bm_blk, 1), lambda i: (i, 0)),
         pl.BlockSpec((D, D), lambda i: (0, 0)),
         pl.BlockSpec((1, D), lambda i: (0, 0)),
         pl.BlockSpec((D, D), lambda i: (0, 0)),
         pl.BlockSpec((1, D), lambda i: (0, 0)),
         pl.BlockSpec((1, D), lambda i: (0, 0))],
        pl.BlockSpec((bm_blk, D), lambda i: (i, 0)),
        jax.ShapeDtypeStruct((M, D), jnp.float32),
    )(x, s, sang.reshape(M, 1), deg.reshape(M, 1), ws, bs.reshape(1, D),
      wm[:D], wm[D:D + 1], bmv.reshape(1, D))


def _edge1_body(gs_ref, gd_ref, w_ref, wsum_ref, wew_ref, be_ref, o_ref):
    g = gs_ref[...] + gd_ref[...]
    y = jnp.dot(g, wsum_ref[...], preferred_element_type=jnp.float32)
    y = y + 2.0 * (w_ref[...] * wew_ref[...]) + 2.0 * be_ref[...]
    o_ref[...] = _leaky(y)


def _edge1(gs, gd, w, we, be, bm=640):
    M = gs.shape[0]
    wsum = we[:D] + we[D:2 * D]
    return _pc(
        _edge1_body, (M // bm,),
        [pl.BlockSpec((bm, D), lambda i: (i, 0)),
         pl.BlockSpec((bm, D), lambda i: (i, 0)),
         pl.BlockSpec((bm, 1), lambda i: (i, 0)),
         pl.BlockSpec((D, D), lambda i: (0, 0)),
         pl.BlockSpec((1, D), lambda i: (0, 0)),
         pl.BlockSpec((1, D), lambda i: (0, 0))],
        pl.BlockSpec((bm, D), lambda i: (i, 0)),
        jax.ShapeDtypeStruct((M, D), jnp.float32),
    )(gs, gd, w.reshape(M, 1), wsum, we[2 * D:2 * D + 1], be.reshape(1, D))


def _edge2_body(gs_ref, gd_ref, w_ref, ef1_ref, wsum_ref, wew_ref, be_ref,
                wct_ref, wcb_ref, bc_ref, o_ref):
    g = gs_ref[...] + gd_ref[...]
    t = jnp.dot(g, wsum_ref[...], preferred_element_type=jnp.float32)
    t = t + 2.0 * (w_ref[...] * wew_ref[...]) + 2.0 * be_ref[...]
    y = jnp.dot(t, wct_ref[...], preferred_element_type=jnp.float32)
    y = y + jnp.dot(ef1_ref[...], wcb_ref[...], preferred_element_type=jnp.float32)
    o_ref[...] = _leaky(y + bc_ref[...])


def _edge2(gs, gd, w, ef1, we, be, wc, bc, bm=640):
    M = gs.shape[0]
    wsum = we[:D] + we[D:2 * D]
    return _pc(
        _edge2_body, (M // bm,),
        [pl.BlockSpec((bm, D), lambda i: (i, 0)),
         pl.BlockSpec((bm, D), lambda i: (i, 0)),
         pl.BlockSpec((bm, 1), lambda i: (i, 0)),
         pl.BlockSpec((bm, D), lambda i: (i, 0)),
         pl.BlockSpec((D, D), lambda i: (0, 0)),
         pl.BlockSpec((1, D), lambda i: (0, 0)),
         pl.BlockSpec((1, D), lambda i: (0, 0)),
         pl.BlockSpec((D, D), lambda i: (0, 0)),
         pl.BlockSpec((D, D), lambda i: (0, 0)),
         pl.BlockSpec((1, D), lambda i: (0, 0))],
        pl.BlockSpec((bm, D), lambda i: (i, 0)),
        jax.ShapeDtypeStruct((M, D), jnp.float32),
    )(gs, gd, w.reshape(M, 1), ef1, wsum, we[2 * D:2 * D + 1],
      be.reshape(1, D), wc[:D], wc[D:2 * D], bc.reshape(1, D))


def _head_body(ef_ref, e1d_ref, w_ref, wp1a_ref, wp1b_ref, wp1c_ref, bp1_ref,
               wp2_ref, bp2_ref, wq1a_ref, wq1b_ref, wq1c_ref, bq1_ref,
               wq2_ref, bq2_ref, p_ref, q_ref, v_ref):
    ef = ef_ref[...]
    e1d = e1d_ref[...]
    wv = w_ref[...]
    hp = jnp.dot(ef, wp1a_ref[...], preferred_element_type=jnp.float32)
    hp = hp + jnp.dot(e1d, wp1b_ref[...], preferred_element_type=jnp.float32)
    hp = hp + wv * wp1c_ref[...] + bp1_ref[...]
    zp = jnp.dot(hp, wp2_ref[...], preferred_element_type=jnp.float32) + bp2_ref[...]
    sg = jax.nn.sigmoid(zp)
    mx = jnp.max(sg, axis=1, keepdims=True)
    ez = jnp.exp(sg - mx)
    p = ez / jnp.sum(ez, axis=1, keepdims=True)
    hq = jnp.dot(ef, wq1a_ref[...], preferred_element_type=jnp.float32)
    hq = hq + jnp.dot(e1d, wq1b_ref[...], preferred_element_type=jnp.float32)
    hq = hq + wv * wq1c_ref[...] + bq1_ref[...]
    q = jnp.dot(hq, wq2_ref[...], preferred_element_type=jnp.float32) + bq2_ref[...]
    p_ref[...] = p
    q_ref[...] = q
    v_ref[...] = jnp.sum(q * p, axis=1, keepdims=True)


def _head(ef, e1d, w, wp1, bp1, wp2, bp2, wq1, bq1, wq2, bq2, bm=640):
    M = ef.shape[0]
    F = e1d.shape[1]
    C = wp2.shape[1]
    full = lambda r, c: pl.BlockSpec((r, c), lambda i: (0, 0))
    blk = lambda c: pl.BlockSpec((bm, c), lambda i: (i, 0))
    return _pc(
        _head_body, (M // bm,),
        [blk(D), blk(F), blk(1),
         full(D, 256), full(F, 256), full(1, 256), full(1, 256),
         full(256, C), full(1, C),
         full(D, 256), full(F, 256), full(1, 256), full(1, 256),
         full(256, C), full(1, C)],
        [blk(C), blk(C), blk(1)],
        [jax.ShapeDtypeStruct((M, C), jnp.float32),
         jax.ShapeDtypeStruct((M, C), jnp.float32),
         jax.ShapeDtypeStruct((M, 1), jnp.float32)],
    )(ef, e1d, w.reshape(M, 1),
      wp1[:D], wp1[D:D + F], wp1[D + F:D + F + 1], bp1.reshape(1, 256),
      wp2, bp2.reshape(1, C),
      wq1[:D], wq1[D:D + F], wq1[D + F:D + F + 1], bq1.reshape(1, 256),
      wq2, bq2.reshape(1, C))


def _im2col3x3(img_hwc):
    p = jnp.pad(img_hwc, ((1, 1), (1, 1), (0, 0)))
    cols = [p[ky:ky + H, kx:kx + W, :] for ky in range(3) for kx in range(3)]
    return jnp.concatenate(cols, axis=-1).reshape(HW, 9 * img_hwc.shape[2])


def kernel(edge_weights, img1, img2, sp_indices, edge_index, angles,
           edge_features_1d, conv1_w, conv1_b, conv2_w, conv2_b,
           Ws1, bs1, Wm1, bm1, We1, be1,
           Ws2, bs2, Wm2, bm2, We2, be2, Wc2, bc2,
           Wp1, bp1, Wp2, bp2, Wq1, bq1, Wq2, bq2):
    f32 = jnp.float32
    imgs = jnp.stack([img1, img2], axis=-1)
    X1 = _im2col3x3(imgs)
    W1 = conv1_w.transpose(2, 3, 1, 0).reshape(18, 64)
    h = _mm_leaky(X1, W1, conv1_b, bm=1024)
    X2 = _im2col3x3(h.reshape(H, W, 64))
    W2 = conv2_w.transpose(2, 3, 1, 0).reshape(576, D)
    pix = _mm_leaky(X2, W2, conv2_b, bm=1024)

    sp = sp_indices.astype(jnp.int32)
    S_sp = jax.ops.segment_sum(pix, sp, num_segments=NP)
    cnt_sp = jax.ops.segment_sum(jnp.ones((HW,), f32), sp, num_segments=NP)
    nf0 = _segdiv(S_sp, cnt_sp, bm=512)

    src = edge_index[0].astype(jnp.int32)
    dst = edge_index[1].astype(jnp.int32)
    ones_e = jnp.ones((2 * E,), f32)
    deg = jax.ops.segment_sum(ones_e, dst, num_segments=NP)
    sang = jax.ops.segment_sum(angles, dst, num_segments=NP)

    s_idx = src[:E]
    d_idx = dst[:E]

    S1 = jax.ops.segment_sum(nf0[src], dst, num_segments=NP)
    nf1 = _node_update(nf0, S1, sang, deg, Ws1, bs1, Wm1, bm1)
    ef1 = _edge1(nf1[s_idx], nf1[d_idx], edge_weights, We1, be1)

    S2 = jax.ops.segment_sum(nf1[src], dst, num_segments=NP)
    nf2 = _node_update(nf1, S2, sang, deg, Ws2, bs2, Wm2, bm2)
    efc = _edge2(nf2[s_idx], nf2[d_idx], edge_weights, ef1, We2, be2, Wc2, bc2)

    p, q, v = _head(efc, edge_features_1d, edge_weights,
                    Wp1, bp1, Wp2, bp2, Wq1, bq1, Wq2, bq2)
    return (p, q, v.reshape(E))


# speedup vs baseline: 1.1454x; 1.1454x over previous
import functools
import jax
import jax.numpy as jnp
from jax import lax
from jax.experimental import pallas as pl
from jax.experimental.pallas import tpu as pltpu

try:
    from jax.experimental.pallas import tpu_sc as plsc
    _info = plsc.get_sparse_core_info()
    NC, NS = _info.num_cores, _info.num_subcores
except Exception:
    plsc = None
    NC, NS = 2, 16

H = 384
W = 384
HW = H * W
D = 256
NP = 10112
E = 80000
EP = 163840
NEG = 0.01
NW = NC * NS
CK = 128

_INTERPRET = False


def _leaky(x):
    return jnp.where(x >= 0, x, NEG * x)


def _pc(body, grid, in_specs, out_specs, out_shape):
    return pl.pallas_call(body, grid=grid, in_specs=in_specs,
                          out_specs=out_specs, out_shape=out_shape,
                          interpret=_INTERPRET)


def _make_sc_seg(B, gather_table, emit_gather, with_scatter, aux_ang, aux_cnt,
                 main=True):
    nb = B // NW
    nit = nb // CK
    rpw = NP // NS

    out_type = []
    if with_scatter:
        out_type += [jax.ShapeDtypeStruct((NC, NP, 128), jnp.float32)] * 2
    if emit_gather:
        out_type += [jax.ShapeDtypeStruct((B, 128), jnp.float32)] * 2
    if aux_ang:
        out_type.append(jax.ShapeDtypeStruct((NC, NP, 16), jnp.float32))
    if aux_cnt:
        out_type.append(jax.ShapeDtypeStruct((NC, NP, 16), jnp.float32))

    scratch = [pltpu.VMEM((CK,), jnp.int32),
               pltpu.SemaphoreType.DMA]
    if main:
        scratch.insert(1, pltpu.VMEM((CK, 128), jnp.float32))
    if gather_table:
        scratch.append(pltpu.VMEM((CK,), jnp.int32))
    if aux_ang:
        scratch.append(pltpu.VMEM((CK, 16), jnp.float32))
    if aux_cnt:
        scratch.append(pltpu.VMEM((CK, 16), jnp.float32))
    if with_scatter:
        scratch.append(pltpu.VMEM_SHARED((NP, 128), jnp.float32))
    if aux_ang:
        scratch.append(pltpu.VMEM_SHARED((NP, 16), jnp.float32))
    if aux_cnt:
        scratch.append(pltpu.VMEM_SHARED((NP, 16), jnp.float32))

    mesh = plsc.VectorSubcoreMesh(core_axis_name="c", subcore_axis_name="s",
                                  num_cores=NC)

    @functools.partial(pl.kernel, mesh=mesh, out_type=out_type,
                       scratch_types=scratch)
    def k(*refs):
        it = iter(refs)
        d0 = next(it) if main else None
        d1 = next(it) if main else None
        dst_h = next(it)
        src_h = next(it) if gather_table else None
        ang_h = next(it) if aux_ang else None
        zrow_h = next(it)
        z16_h = next(it)
        ones_h = next(it)
        s0_o = next(it) if with_scatter else None
        s1_o = next(it) if with_scatter else None
        g0_o = next(it) if emit_gather else None
        g1_o = next(it) if emit_gather else None
        anga_o = next(it) if aux_ang else None
        cnta_o = next(it) if aux_cnt else None
        dst_v = next(it)
        rows_v = next(it) if main else None
        sem = next(it)
        src_v = next(it) if gather_table else None
        ang_v = next(it) if aux_ang else None
        ones_v = next(it) if aux_cnt else None
        acc = next(it) if with_scatter else None
        anga = next(it) if aux_ang else None
        cnta = next(it) if aux_cnt else None

        cid = lax.axis_index("c")
        sid = lax.axis_index("s")
        wid = sid * NC + cid
        base0 = wid * nb

        if aux_cnt:
            pltpu.sync_copy(ones_h, ones_v)

        for h in range(2 if main else 1):
            dat = (d0, d1)[h]
            g_o = (g0_o, g1_o)[h] if emit_gather else None
            s_o = (s0_o, s1_o)[h] if with_scatter else None
            do_aux = h == 0 and (aux_ang or aux_cnt)
            any_acc = with_scatter or do_aux

            if with_scatter:
                pltpu.sync_copy(zrow_h, acc.at[pl.ds(sid * rpw, rpw)])
            if do_aux and aux_ang:
                pltpu.sync_copy(z16_h, anga.at[pl.ds(sid * rpw, rpw)])
            if do_aux and aux_cnt:
                pltpu.sync_copy(z16_h, cnta.at[pl.ds(sid * rpw, rpw)])
            if any_acc:
                plsc.subcore_barrier()

            def body(i, carry):
                base = base0 + i * CK
                pltpu.sync_copy(dst_h.at[pl.ds(base, CK)], dst_v)
                if main:
                    if gather_table:
                        pltpu.sync_copy(src_h.at[pl.ds(base, CK)], src_v)
                        pltpu.async_copy(dat.at[src_v], rows_v, sem).wait()
                    else:
                        pltpu.sync_copy(dat.at[pl.ds(base, CK)], rows_v)
                if with_scatter:
                    pltpu.sync_copy(rows_v, acc.at[dst_v], add=True)
                if emit_gather:
                    pltpu.sync_copy(rows_v, g_o.at[pl.ds(base, CK)])
                if do_aux and aux_ang:
                    pltpu.sync_copy(ang_h.at[pl.ds(base, CK)], ang_v)
                    pltpu.sync_copy(ang_v, anga.at[dst_v], add=True)
                if do_aux and aux_cnt:
                    pltpu.sync_copy(ones_v, cnta.at[dst_v], add=True)
                return carry

            lax.fori_loop(0, nit, body, 0)

            if any_acc:
                plsc.subcore_barrier()
            if with_scatter:
                pltpu.sync_copy(acc.at[pl.ds(sid * rpw, rpw)],
                                s_o.at[cid, pl.ds(sid * rpw, rpw)])
            if do_aux and aux_ang:
                pltpu.sync_copy(anga.at[pl.ds(sid * rpw, rpw)],
                                anga_o.at[cid, pl.ds(sid * rpw, rpw)])
            if do_aux and aux_cnt:
                pltpu.sync_copy(cnta.at[pl.ds(sid * rpw, rpw)],
                                cnta_o.at[cid, pl.ds(sid * rpw, rpw)])
            if any_acc:
                plsc.subcore_barrier()

    def run(dst, d0=None, d1=None, src=None, ang=None):
        zrow = jnp.zeros((rpw, 128), jnp.float32)
        z16 = jnp.zeros((rpw, 16), jnp.float32)
        ones = jnp.ones((CK, 16), jnp.float32)
        args = [d0, d1] if main else []
        args.append(dst)
        if gather_table:
            args.append(src)
        if aux_ang:
            args.append(ang)
        args += [zrow, z16, ones]
        return k(*args)

    return run


def _mm_leaky_body(x_ref, w_ref, b_ref, o_ref):
    o_ref[...] = _leaky(jnp.dot(x_ref[...], w_ref[...],
                                preferred_element_type=jnp.float32) + b_ref[...])


def _mm_leaky(x, w, b, bm):
    M, K = x.shape
    N = w.shape[1]
    return _pc(
        _mm_leaky_body, (M // bm,),
        [pl.BlockSpec((bm, K), lambda i: (i, 0)),
         pl.BlockSpec((K, N), lambda i: (0, 0)),
         pl.BlockSpec((1, N), lambda i: (0, 0))],
        pl.BlockSpec((bm, N), lambda i: (i, 0)),
        jax.ShapeDtypeStruct((M, N), jnp.float32),
    )(x, w, b.reshape(1, N))


def _mm_leaky2_body(x_ref, w_ref, b_ref, o0_ref, o1_ref):
    y = _leaky(jnp.dot(x_ref[...], w_ref[...],
                       preferred_element_type=jnp.float32) + b_ref[...])
    o0_ref[...] = y[:, :128]
    o1_ref[...] = y[:, 128:]


def _mm_leaky2(x, w, b, bm):
    M, K = x.shape
    N = w.shape[1]
    blk = pl.BlockSpec((bm, 128), lambda i: (i, 0))
    return _pc(
        _mm_leaky2_body, (M // bm,),
        [pl.BlockSpec((bm, K), lambda i: (i, 0)),
         pl.BlockSpec((K, N), lambda i: (0, 0)),
         pl.BlockSpec((1, N), lambda i: (0, 0))],
        [blk, blk],
        [jax.ShapeDtypeStruct((M, 128), jnp.float32)] * 2,
    )(x, w, b.reshape(1, N))


def _segdiv2_body(s0_ref, s1_ref, c_ref, o0_ref, o1_ref):
    c = jnp.maximum(c_ref[...], 1.0)
    o0_ref[...] = (s0_ref[0] + s0_ref[1]) / c
    o1_ref[...] = (s1_ref[0] + s1_ref[1]) / c


def _segdiv2(s0p, s1p, cnt, bm=632):
    M = s0p.shape[1]
    pblk = pl.BlockSpec((NC, bm, 128), lambda i: (0, i, 0))
    blk = pl.BlockSpec((bm, 128), lambda i: (i, 0))
    return _pc(
        _segdiv2_body, (M // bm,),
        [pblk, pblk, pl.BlockSpec((bm, 1), lambda i: (i, 0))],
        [blk, blk],
        [jax.ShapeDtypeStruct((M, 128), jnp.float32)] * 2,
    )(s0p, s1p, cnt.reshape(M, 1))


def _node_body(x0_ref, x1_ref, s0_ref, s1_ref, ang_ref, cnt_ref, ws_ref,
               bs_ref, wmx_ref, wma_ref, bmv_ref, o0_ref, o1_ref):
    deg = cnt_ref[...]
    sang = ang_ref[...]
    agg = jnp.dot(s0_ref[0] + s0_ref[1], wmx_ref[:128],
                  preferred_element_type=jnp.float32)
    agg = agg + jnp.dot(s1_ref[0] + s1_ref[1], wmx_ref[128:],
                        preferred_element_type=jnp.float32)
    agg = agg + sang * wma_ref[...] + deg * bmv_ref[...]
    agg = agg / jnp.maximum(deg, 1.0)
    y = jnp.dot(x0_ref[...], ws_ref[:128], preferred_element_type=jnp.float32)
    y = y + jnp.dot(x1_ref[...], ws_ref[128:], preferred_element_type=jnp.float32)
    y = _leaky(y + bs_ref[...] + agg)
    o0_ref[...] = y[:, :128]
    o1_ref[...] = y[:, 128:]


def _node_update(x0, x1, s0p, s1p, angp, cntp, ws, bs, wm, bmv, bm=632):
    M = x0.shape[0]
    blk = pl.BlockSpec((bm, 128), lambda i: (i, 0))
    pblk = pl.BlockSpec((NC, bm, 128), lambda i: (0, i, 0))
    ablk = pl.BlockSpec((bm, 1), lambda i: (i, 0))
    full = lambda r, c: pl.BlockSpec((r, c), lambda i: (0, 0))
    return _pc(
        _node_body, (M // bm,),
        [blk, blk, pblk, pblk, ablk, ablk,
         full(D, D), full(1, D), full(D, D), full(1, D), full(1, D)],
        [blk, blk],
        [jax.ShapeDtypeStruct((M, 128), jnp.float32)] * 2,
    )(x0, x1, s0p, s1p, angp.reshape(M, 1), cntp.reshape(M, 1), ws,
      bs.reshape(1, D), wm[:D], wm[D:D + 1], bmv.reshape(1, D))


def _edge1_body(gs0_ref, gd0_ref, gs1_ref, gd1_ref, w_ref, wsum_ref, wew_ref,
                be_ref, o_ref):
    y = jnp.dot(gs0_ref[...] + gd0_ref[...], wsum_ref[:128],
                preferred_element_type=jnp.float32)
    y = y + jnp.dot(gs1_ref[...] + gd1_ref[...], wsum_ref[128:],
                    preferred_element_type=jnp.float32)
    y = y + 2.0 * (w_ref[...] * wew_ref[...]) + 2.0 * be_ref[...]
    o_ref[...] = _leaky(y)


def _edge1(g0, g1, w, we, be, bm=640):
    nsb = E // bm
    wsum = we[:D] + we[D:2 * D]
    sblk = pl.BlockSpec((bm, 128), lambda i: (i, 0))
    dblk = pl.BlockSpec((bm, 128), lambda i: (i + nsb, 0))
    full = lambda r, c: pl.BlockSpec((r, c), lambda i: (0, 0))
    return _pc(
        _edge1_body, (nsb,),
        [sblk, dblk, sblk, dblk, pl.BlockSpec((bm, 1), lambda i: (i, 0)),
         full(D, D), full(1, D), full(1, D)],
        pl.BlockSpec((bm, D), lambda i: (i, 0)),
        jax.ShapeDtypeStruct((E, D), jnp.float32),
    )(g0, g0, g1, g1, w.reshape(E, 1), wsum, we[2 * D:2 * D + 1],
      be.reshape(1, D))


def _edge2_body(gs0_ref, gd0_ref, gs1_ref, gd1_ref, w_ref, ef1_ref, wsum_ref,
                wew_ref, be_ref, wct_ref, wcb_ref, bc_ref, o_ref):
    t = jnp.dot(gs0_ref[...] + gd0_ref[...], wsum_ref[:128],
                preferred_element_type=jnp.float32)
    t = t + jnp.dot(gs1_ref[...] + gd1_ref[...], wsum_ref[128:],
                    preferred_element_type=jnp.float32)
    t = t + 2.0 * (w_ref[...] * wew_ref[...]) + 2.0 * be_ref[...]
    y = jnp.dot(t, wct_ref[...], preferred_element_type=jnp.float32)
    y = y + jnp.dot(ef1_ref[...], wcb_ref[...], preferred_element_type=jnp.float32)
    o_ref[...] = _leaky(y + bc_ref[...])


def _edge2(g0, g1, w, ef1, we, be, wc, bc, bm=640):
    nsb = E // bm
    wsum = we[:D] + we[D:2 * D]
    sblk = pl.BlockSpec((bm, 128), lambda i: (i, 0))
    dblk = pl.BlockSpec((bm, 128), lambda i: (i + nsb, 0))
    full = lambda r, c: pl.BlockSpec((r, c), lambda i: (0, 0))
    return _pc(
        _edge2_body, (nsb,),
        [sblk, dblk, sblk, dblk, pl.BlockSpec((bm, 1), lambda i: (i, 0)),
         pl.BlockSpec((bm, D), lambda i: (i, 0)),
         full(D, D), full(1, D), full(1, D),
         full(D, D), full(D, D), full(1, D)],
        pl.BlockSpec((bm, D), lambda i: (i, 0)),
        jax.ShapeDtypeStruct((E, D), jnp.float32),
    )(g0, g0, g1, g1, w.reshape(E, 1), ef1, wsum, we[2 * D:2 * D + 1],
      be.reshape(1, D), wc[:D], wc[D:2 * D], bc.reshape(1, D))


def _head_body(ef_ref, e1d_ref, w_ref, wp1a_ref, wp1b_ref, wp1c_ref, bp1_ref,
               wp2_ref, bp2_ref, wq1a_ref, wq1b_ref, wq1c_ref, bq1_ref,
               wq2_ref, bq2_ref, p_ref, q_ref, v_ref):
    ef = ef_ref[...]
    e1d = e1d_ref[...]
    wv = w_ref[...]
    hp = jnp.dot(ef, wp1a_ref[...], preferred_element_type=jnp.float32)
    hp = hp + jnp.dot(e1d, wp1b_ref[...], preferred_element_type=jnp.float32)
    hp = hp + wv * wp1c_ref[...] + bp1_ref[...]
    zp = jnp.dot(hp, wp2_ref[...], preferred_element_type=jnp.float32) + bp2_ref[...]
    sg = jax.nn.sigmoid(zp)
    mx = jnp.max(sg, axis=1, keepdims=True)
    ez = jnp.exp(sg - mx)
    p = ez / jnp.sum(ez, axis=1, keepdims=True)
    hq = jnp.dot(ef, wq1a_ref[...], preferred_element_type=jnp.float32)
    hq = hq + jnp.dot(e1d, wq1b_ref[...], preferred_element_type=jnp.float32)
    hq = hq + wv * wq1c_ref[...] + bq1_ref[...]
    q = jnp.dot(hq, wq2_ref[...], preferred_element_type=jnp.float32) + bq2_ref[...]
    p_ref[...] = p
    q_ref[...] = q
    v_ref[...] = jnp.sum(q * p, axis=1, keepdims=True)


def _head(ef, e1d, w, wp1, bp1, wp2, bp2, wq1, bq1, wq2, bq2, bm=640):
    M = ef.shape[0]
    F = e1d.shape[1]
    C = wp2.shape[1]
    full = lambda r, c: pl.BlockSpec((r, c), lambda i: (0, 0))
    blk = lambda c: pl.BlockSpec((bm, c), lambda i: (i, 0))
    return _pc(
        _head_body, (M // bm,),
        [blk(D), blk(F), blk(1),
         full(D, 256), full(F, 256), full(1, 256), full(1, 256),
         full(256, C), full(1, C),
         full(D, 256), full(F, 256), full(1, 256), full(1, 256),
         full(256, C), full(1, C)],
        [blk(C), blk(C), blk(1)],
        [jax.ShapeDtypeStruct((M, C), jnp.float32),
         jax.ShapeDtypeStruct((M, C), jnp.float32),
         jax.ShapeDtypeStruct((M, 1), jnp.float32)],
    )(ef, e1d, w.reshape(M, 1),
      wp1[:D], wp1[D:D + F], wp1[D + F:D + F + 1], bp1.reshape(1, 256),
      wp2, bp2.reshape(1, C),
      wq1[:D], wq1[D:D + F], wq1[D + F:D + F + 1], bq1.reshape(1, 256),
      wq2, bq2.reshape(1, C))


def _im2col3x3(img_hwc):
    p = jnp.pad(img_hwc, ((1, 1), (1, 1), (0, 0)))
    cols = [p[ky:ky + H, kx:kx + W, :] for ky in range(3) for kx in range(3)]
    return jnp.concatenate(cols, axis=-1).reshape(HW, 9 * img_hwc.shape[2])


def kernel(edge_weights, img1, img2, sp_indices, edge_index, angles,
           edge_features_1d, conv1_w, conv1_b, conv2_w, conv2_b,
           Ws1, bs1, Wm1, bm1, We1, be1,
           Ws2, bs2, Wm2, bm2, We2, be2, Wc2, bc2,
           Wp1, bp1, Wp2, bp2, Wq1, bq1, Wq2, bq2):
    f32 = jnp.float32
    imgs = jnp.stack([img1, img2], axis=-1)
    X1 = _im2col3x3(imgs)
    W1 = conv1_w.transpose(2, 3, 1, 0).reshape(18, 64)
    h = _mm_leaky(X1, W1, conv1_b, bm=1024)
    X2 = _im2col3x3(h.reshape(H, W, 64))
    W2 = conv2_w.transpose(2, 3, 1, 0).reshape(576, D)
    pix0, pix1 = _mm_leaky2(X2, W2, conv2_b, bm=1024)

    sp = sp_indices.astype(jnp.int32)
    src = edge_index[0].astype(jnp.int32)
    dst = edge_index[1].astype(jnp.int32)
    pad_i = jnp.full((EP - 2 * E,), NP - 1, jnp.int32)
    src_p = jnp.concatenate([src, pad_i])
    dst_p = jnp.concatenate([dst, pad_i])

    sc_pool = _make_sc_seg(HW, gather_table=False, emit_gather=False,
                           with_scatter=True, aux_ang=False, aux_cnt=False)
    sc_l1 = _make_sc_seg(EP, gather_table=True, emit_gather=False,
                         with_scatter=True, aux_ang=False, aux_cnt=False)
    sc_l2 = _make_sc_seg(EP, gather_table=True, emit_gather=True,
                         with_scatter=True, aux_ang=False, aux_cnt=False)
    sc_g = _make_sc_seg(EP, gather_table=True, emit_gather=True,
                        with_scatter=False, aux_ang=False, aux_cnt=False)

    Sp0, Sp1 = sc_pool(sp, pix0, pix1)
    cnt_sp = jax.ops.segment_sum(jnp.ones((HW,), f32), sp, num_segments=NP)
    nf0_0, nf0_1 = _segdiv2(Sp0, Sp1, cnt_sp)

    deg = jax.ops.segment_sum(jnp.ones((2 * E,), f32), dst, num_segments=NP)
    sang = jax.ops.segment_sum(angles, dst, num_segments=NP)
    S10, S11 = sc_l1(dst_p, nf0_0, nf0_1, src=src_p)
    nf1_0, nf1_1 = _node_update(nf0_0, nf0_1, S10, S11, sang, deg,
                                Ws1, bs1, Wm1, bm1)

    S20, S21, xg0, xg1 = sc_l2(dst_p, nf1_0, nf1_1, src=src_p)
    ef1 = _edge1(xg0, xg1, edge_weights, We1, be1)
    nf2_0, nf2_1 = _node_update(nf1_0, nf1_1, S20, S21, sang, deg,
                                Ws2, bs2, Wm2, bm2)

    yg0, yg1 = sc_g(dst_p, nf2_0, nf2_1, src=src_p)
    efc = _edge2(yg0, yg1, edge_weights, ef1, We2, be2, Wc2, bc2)

    p, q, v = _head(efc, edge_features_1d, edge_weights,
                    Wp1, bp1, Wp2, bp2, Wq1, bq1, Wq2, bq2)
    return (p, q, v.reshape(E))
